# baseline (speedup 1.0000x reference)
import functools

import jax
import jax.numpy as jnp
from jax import lax
from jax.experimental import pallas as pl
from jax.experimental.pallas import tpu as pltpu
from jax.experimental.pallas import tpu_sc as plsc

_SEQ = 2048
_D = 32
_TAB_ROWS = 255
_NUM_WORKERS = 32
_RPW = _SEQ // _NUM_WORKERS
_F_ROWS = 4096
_CHUNK = _F_ROWS // 16
_CHUNK_B = _CHUNK + 3
_ROW_W = _SEQ * _D
_FK_W = _F_ROWS * _D
_L = 16


def _sc_body(table_hbm, out_hbm, tab_ref, chunk_ref, f_spmem, csem, osem):
    cid = lax.axis_index("c")
    sid = lax.axis_index("s")
    wid = sid * 2 + cid
    i0 = wid * _RPW

    pltpu.sync_copy(table_hbm, tab_ref)

    c254_a = tab_ref[pl.ds(254 * _D, _L)]
    c254_b = tab_ref[pl.ds(254 * _D + _L, _L)]
    c0_a = tab_ref[pl.ds(0, _L)]
    c0_b = tab_ref[pl.ds(_L, _L)]

    m0 = sid * _CHUNK
    m1 = m0 + _CHUNK_B
    e_pre = jnp.clip(1921, m0, m1)
    e_band = jnp.clip(2174, m0, m1)

    def fill_const_pre(m, _):
        o = (m - m0) * _D
        chunk_ref[pl.ds(o, _L)] = c254_a
        chunk_ref[pl.ds(o + _L, _L)] = c254_b
        return _

    def fill_band(m, _):
        o = (m - m0) * _D
        src = (2174 - m) * _D
        chunk_ref[pl.ds(o, _L)] = tab_ref[pl.ds(src, _L)]
        chunk_ref[pl.ds(o + _L, _L)] = tab_ref[pl.ds(src + _L, _L)]
        return _

    def fill_const_post(m, _):
        o = (m - m0) * _D
        chunk_ref[pl.ds(o, _L)] = c0_a
        chunk_ref[pl.ds(o + _L, _L)] = c0_b
        return _

    lax.fori_loop(m0, e_pre, fill_const_pre, 0)
    lax.fori_loop(e_pre, e_band, fill_band, 0)
    lax.fori_loop(e_band, m1, fill_const_post, 0)

    fills = [
        pltpu.async_copy(
            chunk_ref.at[pl.ds(k * _D, _CHUNK * _D)],
            f_spmem.at[pl.ds(k * _FK_W + m0 * _D, _CHUNK * _D)],
            csem,
        )
        for k in range(4)
    ]
    for cp in fills:
        cp.wait()
    plsc.subcore_barrier()

    i0_w = i0 * _D
    for g in range(0, _RPW, 8):
        copies = []
        for r in range(g, g + 8):
            k = (3 - r) % 4
            src = k * _FK_W + (2047 - r - k) * _D - i0_w
            copies.append(
                pltpu.async_copy(
                    f_spmem.at[pl.ds(src, _ROW_W)],
                    out_hbm.at[pl.ds(i0 * _ROW_W + r * _ROW_W, _ROW_W)],
                    osem,
                )
            )
        for cp in copies:
            cp.wait()


def kernel(seq_len, rel_pos_emb):
    del seq_len
    mesh = plsc.VectorSubcoreMesh(core_axis_name="c", subcore_axis_name="s")
    run = functools.partial(
        pl.kernel,
        mesh=mesh,
        out_type=jax.ShapeDtypeStruct((_SEQ * _SEQ * _D,), jnp.float32),
        scratch_types=[
            pltpu.VMEM((_TAB_ROWS * _D,), jnp.float32),
            pltpu.VMEM((_CHUNK_B * _D,), jnp.float32),
            pltpu.VMEM_SHARED((4 * _FK_W,), jnp.float32),
            pltpu.SemaphoreType.DMA,
            pltpu.SemaphoreType.DMA,
        ],
    )(_sc_body)
    flat = run(rel_pos_emb.reshape(-1))
    return flat.reshape(_SEQ, _SEQ, _D)
